# baseline (speedup 1.0000x reference)
import jax
import jax.numpy as jnp
from jax import lax
from jax.experimental import pallas as pl
from jax.experimental.pallas import tpu as pltpu
from jax.experimental.pallas import tpu_sc as plsc

VOCAB = 1000000
EMBED_DIM = 32
BATCH = 16384

_INFO = plsc.get_sparse_core_info()
_NC = _INFO.num_cores
_NS = _INFO.num_subcores
_NW = _NC * _NS
_B_PER_W = BATCH // _NW


def _triplet_gather(a_hbm, p_hbm, n_hbm, table_hbm,
                    out_a, out_p, out_n,
                    ia_v, ip_v, in_v, ra_v, rp_v, rn_v,
                    sem_a, sem_p, sem_n):
    wid = lax.axis_index("s") * _NC + lax.axis_index("c")
    base = wid * _B_PER_W
    sl = pl.ds(base, _B_PER_W)
    pltpu.sync_copy(a_hbm.at[sl], ia_v)
    pltpu.sync_copy(p_hbm.at[sl], ip_v)
    pltpu.sync_copy(n_hbm.at[sl], in_v)
    ca = pltpu.async_copy(table_hbm.at[ia_v], ra_v, sem_a)
    cp = pltpu.async_copy(table_hbm.at[ip_v], rp_v, sem_p)
    cn = pltpu.async_copy(table_hbm.at[in_v], rn_v, sem_n)
    ca.wait()
    pltpu.sync_copy(ra_v, out_a.at[sl])
    cp.wait()
    pltpu.sync_copy(rp_v, out_p.at[sl])
    cn.wait()
    pltpu.sync_copy(rn_v, out_n.at[sl])


@jax.jit
def kernel(anchor, positive, negative, W):
    mesh = plsc.VectorSubcoreMesh(core_axis_name="c", subcore_axis_name="s")
    out = jax.ShapeDtypeStruct((BATCH, EMBED_DIM), jnp.float32)
    f = pl.kernel(
        _triplet_gather,
        mesh=mesh,
        out_type=(out, out, out),
        scratch_types=[
            pltpu.VMEM((_B_PER_W,), jnp.int32),
            pltpu.VMEM((_B_PER_W,), jnp.int32),
            pltpu.VMEM((_B_PER_W,), jnp.int32),
            pltpu.VMEM((_B_PER_W, EMBED_DIM), jnp.float32),
            pltpu.VMEM((_B_PER_W, EMBED_DIM), jnp.float32),
            pltpu.VMEM((_B_PER_W, EMBED_DIM), jnp.float32),
            pltpu.SemaphoreType.DMA,
            pltpu.SemaphoreType.DMA,
            pltpu.SemaphoreType.DMA,
        ],
        compiler_params=pltpu.CompilerParams(use_tc_tiling_on_sc=False),
    )
    return f(anchor, positive, negative, W)
